# baseline (speedup 1.0000x reference)
import functools

import jax
import jax.numpy as jnp
from jax.experimental import pallas as pl


def _smoothness_body(*refs, inv_n, nstreams):
    a_refs = refs[:nstreams]
    z_ref = refs[nstreams]
    zi_refs = refs[nstreams + 1 : 2 * nstreams + 1]
    out_ref = refs[2 * nstreams + 1]
    i = pl.program_id(0)
    zfull = z_ref[...]
    contrib = jnp.float32(0.0)
    for a_ref, zi_ref in zip(a_refs, zi_refs):
        a = a_ref[...]
        zi = zi_ref[...]
        y = jnp.dot(a, zfull, preferred_element_type=jnp.float32)
        d = jnp.sum(a, axis=1)
        s = jnp.sum(zi * zi, axis=1)
        contrib += jnp.sum(d * s) - jnp.sum(zi * y)
    contrib = jnp.reshape(contrib * inv_n, (1, 1))

    @pl.when(i == 0)
    def _init():
        out_ref[...] = contrib

    @pl.when(i != 0)
    def _acc():
        out_ref[...] += contrib


@jax.jit
def kernel(z, coords, precomputed_adj):
    del coords
    n, dim = z.shape
    blk = 512
    ns = 2
    grid = (n // (ns * blk),)

    def a_map(k):
        return lambda i: (ns * i + k, 0)

    out = pl.pallas_call(
        functools.partial(_smoothness_body, inv_n=1.0 / n, nstreams=ns),
        grid=grid,
        in_specs=(
            [pl.BlockSpec((blk, n), a_map(k)) for k in range(ns)]
            + [pl.BlockSpec((n, dim), lambda i: (0, 0))]
            + [pl.BlockSpec((blk, dim), a_map(k)) for k in range(ns)]
        ),
        out_specs=pl.BlockSpec((1, 1), lambda i: (0, 0)),
        out_shape=jax.ShapeDtypeStruct((1, 1), jnp.float32),
    )(*([precomputed_adj] * ns), z, *([z] * ns))
    return out[0, 0]


# speedup vs baseline: 1.1337x; 1.1337x over previous
import functools

import jax
import jax.numpy as jnp
from jax.experimental import pallas as pl


def _smoothness_body(a0_ref, a1_ref, z_ref, out_ref, *, inv_n, blk):
    i = pl.program_id(0)
    zfull = z_ref[...]
    contrib = jnp.float32(0.0)
    for k, a_ref in enumerate((a0_ref, a1_ref)):
        a = a_ref[...]
        zi = z_ref[pl.ds((2 * i + k) * blk, blk), :]
        y = jnp.dot(a, zfull, preferred_element_type=jnp.float32)
        d = jnp.sum(a, axis=1)
        s = jnp.sum(zi * zi, axis=1)
        contrib += jnp.sum(d * s) - jnp.sum(zi * y)
    contrib = jnp.reshape(contrib * inv_n, (1, 1))

    @pl.when(i == 0)
    def _init():
        out_ref[...] = contrib

    @pl.when(i != 0)
    def _acc():
        out_ref[...] += contrib


@jax.jit
def kernel(z, coords, precomputed_adj):
    del coords
    n, dim = z.shape
    blk = 256
    grid = (n // (2 * blk),)
    out = pl.pallas_call(
        functools.partial(_smoothness_body, inv_n=1.0 / n, blk=blk),
        grid=grid,
        in_specs=[
            pl.BlockSpec((blk, n), lambda i: (2 * i, 0)),
            pl.BlockSpec((blk, n), lambda i: (2 * i + 1, 0)),
            pl.BlockSpec((n, dim), lambda i: (0, 0)),
        ],
        out_specs=pl.BlockSpec((1, 1), lambda i: (0, 0)),
        out_shape=jax.ShapeDtypeStruct((1, 1), jnp.float32),
    )(precomputed_adj, precomputed_adj, z)
    return out[0, 0]
